# Initial kernel scaffold; baseline (speedup 1.0000x reference)
import jax
import jax.numpy as jnp
from jax.experimental import pallas as pl


def kernel(X, edge_index, W_ff1, b_ff1, W_A, W_B, W_ff2, b_ff2):
    raise NotImplementedError("write your pallas kernel here")


# speedup vs baseline: 7.6312x; 7.6312x over previous
import functools

import jax
import jax.numpy as jnp
from jax.experimental import pallas as pl
from jax.experimental.pallas import tpu as pltpu

_SQRT2 = 1.4142135623730951
_HID = 128
_MIX = 4
_PAY = 5 * _HID


def _gelu(x):
    return 0.5 * x * (1.0 + jax.lax.erf(x / _SQRT2))


def _node_body(x_ref, wff1_ref, bff1_ref, wa_ref, wb_ref, h_ref, cext_ref):
    x = x_ref[...]
    h = _gelu(jnp.dot(x, wff1_ref[...], preferred_element_type=jnp.float32)
              + bff1_ref[...])
    t = _gelu(jnp.dot(_gelu(h), wa_ref[...],
                      preferred_element_type=jnp.float32))
    t = jnp.dot(t, wb_ref[...], preferred_element_type=jnp.float32)
    h_ref[...] = h
    bn = x.shape[0]
    tail = jnp.concatenate(
        [t, jnp.ones((bn, 1), jnp.float32),
         jnp.zeros((bn, _HID - _MIX - 1), jnp.float32)], axis=1)
    cext_ref[...] = jnp.concatenate(
        [h * t[:, m:m + 1] for m in range(_MIX)] + [tail], axis=1)


def _scatter_body(src_ref, dst_ref, cext_ref, m_ref):
    @pl.when(pl.program_id(0) == 0)
    def _():
        m_ref[...] = jnp.zeros(m_ref.shape, m_ref.dtype)

    be = src_ref.shape[-1]

    def body(i, _):
        s = src_ref[0, 0, i]
        d = dst_ref[0, 0, i]
        m_ref[pl.ds(d, 1), :] += cext_ref[pl.ds(s, 1), :]
        return 0

    jax.lax.fori_loop(0, be, body, 0, unroll=False)


def _final_body(m_ref, h_ref, x_ref, wff2_ref, bff2_ref, o_ref):
    m = m_ref[...]
    agg = _gelu(m[:, :_MIX * _HID])
    cnt = m[:, _MIX * _HID + _MIX:_MIX * _HID + _MIX + 1]
    acc = jnp.zeros(h_ref.shape, jnp.float32)
    for mm in range(_MIX):
        s_col = m[:, _MIX * _HID + mm:_MIX * _HID + mm + 1]
        acc = acc + agg[:, mm * _HID:(mm + 1) * _HID] * s_col
    agg2 = acc / jnp.clip(cnt, 1.0, None)
    z = _gelu(jnp.concatenate([agg2, h_ref[...]], axis=1))
    o_ref[...] = (jnp.dot(z, wff2_ref[...], preferred_element_type=jnp.float32)
                  + bff2_ref[...] + x_ref[...])


def _build(n, e, in_dim, out_dim, bn, be, interpret=False):
    nb_e = e // be
    assert nb_e * be == e and n % bn == 0

    node = pl.pallas_call(
        _node_body,
        grid=(n // bn,),
        in_specs=[
            pl.BlockSpec((bn, in_dim), lambda i: (i, 0)),
            pl.BlockSpec((in_dim, _HID), lambda i: (0, 0)),
            pl.BlockSpec((1, _HID), lambda i: (0, 0)),
            pl.BlockSpec((_HID, _HID), lambda i: (0, 0)),
            pl.BlockSpec((_HID, _MIX), lambda i: (0, 0)),
        ],
        out_specs=[
            pl.BlockSpec((bn, _HID), lambda i: (i, 0)),
            pl.BlockSpec((bn, _PAY), lambda i: (i, 0)),
        ],
        out_shape=[
            jax.ShapeDtypeStruct((n, _HID), jnp.float32),
            jax.ShapeDtypeStruct((n, _PAY), jnp.float32),
        ],
        interpret=interpret,
    )

    scatter = pl.pallas_call(
        _scatter_body,
        grid=(nb_e,),
        in_specs=[
            pl.BlockSpec((1, 1, be), lambda i: (i, 0, 0), memory_space=pltpu.SMEM),
            pl.BlockSpec((1, 1, be), lambda i: (i, 0, 0), memory_space=pltpu.SMEM),
            pl.BlockSpec((n, _PAY), lambda i: (0, 0)),
        ],
        out_specs=pl.BlockSpec((n, _PAY), lambda i: (0, 0)),
        out_shape=jax.ShapeDtypeStruct((n, _PAY), jnp.float32),
        compiler_params=pltpu.CompilerParams(
            vmem_limit_bytes=100 * 1024 * 1024),
        interpret=interpret,
    )

    final = pl.pallas_call(
        _final_body,
        grid=(n // bn,),
        in_specs=[
            pl.BlockSpec((bn, _PAY), lambda i: (i, 0)),
            pl.BlockSpec((bn, _HID), lambda i: (i, 0)),
            pl.BlockSpec((bn, in_dim), lambda i: (i, 0)),
            pl.BlockSpec((2 * _HID, out_dim), lambda i: (0, 0)),
            pl.BlockSpec((1, out_dim), lambda i: (0, 0)),
        ],
        out_specs=pl.BlockSpec((bn, out_dim), lambda i: (i, 0)),
        out_shape=jax.ShapeDtypeStruct((n, out_dim), jnp.float32),
        interpret=interpret,
    )

    def run(X, edge_index, W_ff1, b_ff1, W_A, W_B, W_ff2, b_ff2):
        h, cext = node(X, W_ff1, b_ff1.reshape(1, _HID), W_A, W_B)
        src = edge_index[0].reshape(nb_e, 1, be)
        dst = edge_index[1].reshape(nb_e, 1, be)
        m = scatter(src, dst, cext)
        return final(m, h, X, W_ff2, b_ff2.reshape(1, out_dim))

    return run


@jax.jit
def kernel(X, edge_index, W_ff1, b_ff1, W_A, W_B, W_ff2, b_ff2):
    run = _build(10000, 160000, 128, 128, bn=1000, be=2500)
    return run(X, edge_index, W_ff1, b_ff1, W_A, W_B, W_ff2, b_ff2)


# speedup vs baseline: 18.5270x; 2.4278x over previous
import functools

import jax
import jax.numpy as jnp
from jax import lax
from jax.experimental import pallas as pl
from jax.experimental.pallas import tpu as pltpu
from jax.experimental.pallas import tpu_sc as plsc

_SQRT2 = 1.4142135623730951
_HID = 128
_MIX = 4
_PQ = 144
_NP = 10240
_K = 80


def _gelu(x):
    return 0.5 * x * (1.0 + jax.lax.erf(x / _SQRT2))


def _node_body(x_ref, wff1_ref, bff1_ref, wa_ref, wb_ref, h_ref, cq_ref):
    x = x_ref[...]
    h = _gelu(jnp.dot(x, wff1_ref[...], preferred_element_type=jnp.float32)
              + bff1_ref[...])
    t = _gelu(jnp.dot(_gelu(h), wa_ref[...],
                      preferred_element_type=jnp.float32))
    t = jnp.dot(t, wb_ref[...], preferred_element_type=jnp.float32)
    h_ref[...] = h
    bn = x.shape[0]
    zero_tail = jnp.zeros((bn, _PQ - _HID), jnp.float32)
    tail0 = jnp.concatenate(
        [t, jnp.ones((bn, 1), jnp.float32),
         jnp.zeros((bn, _PQ - _HID - _MIX - 1), jnp.float32)], axis=1)
    for q in range(_MIX):
        tail = tail0 if q == 0 else zero_tail
        cq_ref[q] = jnp.concatenate([h * t[:, q:q + 1], tail], axis=1)


def _sc_scatter_body(cq_hbm, src_hbm, dst_hbm, m_hbm,
                     idx_s, idx_d, rows, acc, sem):
    c = lax.axis_index("c")
    s = lax.axis_index("s")
    n_batches = 160000 // 16 // _K

    for p in range(2):
        q = 2 * p + c
        qbase = q * _NP

        def _zrow(r, carry):
            for j in range(_PQ // 16):
                rows[r, pl.ds(j * 16, 16)] = jnp.zeros((16,), jnp.float32)
            return carry
        lax.fori_loop(0, _K, _zrow, 0)
        rows_per_tile = _NP // 16
        for b in range(rows_per_tile // _K):
            pltpu.sync_copy(rows, acc.at[pl.ds(s * rows_per_tile + b * _K, _K)])
        plsc.subcore_barrier()

        def _ebody(b, carry):
            eb = pl.multiple_of(s * 10000 + b * _K, 16)
            pltpu.sync_copy(src_hbm.at[pl.ds(eb, _K)], idx_s)
            for i in range(_K // 16):
                idx_s[pl.ds(i * 16, 16)] = idx_s[pl.ds(i * 16, 16)] + qbase
            pltpu.async_copy(cq_hbm.at[idx_s], rows, sem).wait()
            pltpu.sync_copy(dst_hbm.at[pl.ds(eb, _K)], idx_d)
            pltpu.sync_copy(rows, acc.at[idx_d], add=True)
            return carry
        lax.fori_loop(0, n_batches, _ebody, 0)
        plsc.subcore_barrier()

        pltpu.sync_copy(
            acc.at[pl.ds(s * rows_per_tile, rows_per_tile)],
            m_hbm.at[pl.ds(qbase + s * rows_per_tile, rows_per_tile)])
        plsc.subcore_barrier()


def _final_body(m_ref, h_ref, x_ref, wff2_ref, bff2_ref, o_ref):
    m4 = m_ref[...]
    cnt = m4[0, :, _HID + _MIX:_HID + _MIX + 1]
    acc = jnp.zeros(h_ref.shape, jnp.float32)
    for q in range(_MIX):
        aggq = _gelu(m4[q, :, :_HID])
        sq = m4[0, :, _HID + q:_HID + q + 1]
        acc = acc + aggq * sq
    agg2 = acc / jnp.clip(cnt, 1.0, None)
    z = _gelu(jnp.concatenate([agg2, h_ref[...]], axis=1))
    o_ref[...] = (jnp.dot(z, wff2_ref[...], preferred_element_type=jnp.float32)
                  + bff2_ref[...] + x_ref[...])


def _build(n, e, in_dim, out_dim, bn):
    assert n % bn == 0

    node = pl.pallas_call(
        _node_body,
        grid=(n // bn,),
        in_specs=[
            pl.BlockSpec((bn, in_dim), lambda i: (i, 0)),
            pl.BlockSpec((in_dim, _HID), lambda i: (0, 0)),
            pl.BlockSpec((1, _HID), lambda i: (0, 0)),
            pl.BlockSpec((_HID, _HID), lambda i: (0, 0)),
            pl.BlockSpec((_HID, _MIX), lambda i: (0, 0)),
        ],
        out_specs=[
            pl.BlockSpec((bn, _HID), lambda i: (i, 0)),
            pl.BlockSpec((_MIX, bn, _PQ), lambda i: (0, i, 0)),
        ],
        out_shape=[
            jax.ShapeDtypeStruct((n, _HID), jnp.float32),
            jax.ShapeDtypeStruct((_MIX, _NP, _PQ), jnp.float32),
        ],
    )

    mesh = plsc.VectorSubcoreMesh(
        core_axis_name="c", subcore_axis_name="s", num_cores=2)
    scatter = pl.kernel(
        _sc_scatter_body,
        mesh=mesh,
        compiler_params=pltpu.CompilerParams(use_tc_tiling_on_sc=False),
        out_type=jax.ShapeDtypeStruct((_MIX * _NP, _PQ), jnp.float32),
        scratch_types=[
            pltpu.VMEM((_K,), jnp.int32),
            pltpu.VMEM((_K,), jnp.int32),
            pltpu.VMEM((_K, _PQ), jnp.float32),
            pltpu.VMEM_SHARED((_NP, _PQ), jnp.float32),
            pltpu.SemaphoreType.DMA,
        ],
    )

    final = pl.pallas_call(
        _final_body,
        grid=(n // bn,),
        in_specs=[
            pl.BlockSpec((_MIX, bn, _PQ), lambda i: (0, i, 0)),
            pl.BlockSpec((bn, _HID), lambda i: (i, 0)),
            pl.BlockSpec((bn, in_dim), lambda i: (i, 0)),
            pl.BlockSpec((2 * _HID, out_dim), lambda i: (0, 0)),
            pl.BlockSpec((1, out_dim), lambda i: (0, 0)),
        ],
        out_specs=pl.BlockSpec((bn, out_dim), lambda i: (i, 0)),
        out_shape=jax.ShapeDtypeStruct((n, out_dim), jnp.float32),
    )

    def run(X, edge_index, W_ff1, b_ff1, W_A, W_B, W_ff2, b_ff2):
        h, cq = node(X, W_ff1, b_ff1.reshape(1, _HID), W_A, W_B)
        m = scatter(cq.reshape(_MIX * _NP, _PQ),
                    edge_index[0], edge_index[1])
        return final(m.reshape(_MIX, _NP, _PQ), h, X,
                     W_ff2, b_ff2.reshape(1, out_dim))

    return run


@jax.jit
def kernel(X, edge_index, W_ff1, b_ff1, W_A, W_B, W_ff2, b_ff2):
    run = _build(10000, 160000, 128, 128, bn=1000)
    return run(X, edge_index, W_ff1, b_ff1, W_A, W_B, W_ff2, b_ff2)


# speedup vs baseline: 33.1203x; 1.7877x over previous
import functools

import jax
import jax.numpy as jnp
from jax import lax
from jax.experimental import pallas as pl
from jax.experimental.pallas import tpu as pltpu
from jax.experimental.pallas import tpu_sc as plsc

_SQRT2 = 1.4142135623730951
_HID = 128
_MIX = 4
_PQ = 144
_NP = 10240
_K = 80


def _gelu(x):
    return 0.5 * x * (1.0 + jax.lax.erf(x / _SQRT2))


def _node_body(x_ref, wff1_ref, bff1_ref, wa_ref, wb_ref, h_ref, cq_ref):
    x = x_ref[...]
    h = _gelu(jnp.dot(x, wff1_ref[...], preferred_element_type=jnp.float32)
              + bff1_ref[...])
    t = _gelu(jnp.dot(_gelu(h), wa_ref[...],
                      preferred_element_type=jnp.float32))
    t = jnp.dot(t, wb_ref[...], preferred_element_type=jnp.float32)
    h_ref[...] = h
    bn = x.shape[0]
    zero_tail = jnp.zeros((bn, _PQ - _HID), jnp.float32)
    tail0 = jnp.concatenate(
        [t, jnp.ones((bn, 1), jnp.float32),
         jnp.zeros((bn, _PQ - _HID - _MIX - 1), jnp.float32)], axis=1)
    for q in range(_MIX):
        tail = tail0 if q == 0 else zero_tail
        cq_ref[q] = jnp.concatenate([h * t[:, q:q + 1], tail], axis=1)


def _sc_scatter_body(cq_hbm, ei_hbm, m_hbm,
                     idx0, idx1, rows0, rows1, acc, sem0, sem1):
    c = lax.axis_index("c")
    s = lax.axis_index("s")
    n_batches = 160000 // 16 // _K

    for p in range(2):
        q = 2 * p + c
        qbase = q * _NP

        def _start(b, idx, rows, sem):
            eb = pl.multiple_of(s * 10000 + b * _K, 16)
            pltpu.sync_copy(ei_hbm.at[:, pl.ds(eb, _K)], idx)
            for i in range(_K // 16):
                idx[0, pl.ds(i * 16, 16)] = idx[0, pl.ds(i * 16, 16)] + qbase
            pltpu.async_copy(cq_hbm.at[idx.at[0]], rows, sem)

        def _finish(idx, rows, sem):
            pltpu.make_async_copy(cq_hbm.at[idx.at[0]], rows, sem).wait()
            pltpu.sync_copy(rows, acc.at[idx.at[1]], add=True)

        def _zrow(r, carry):
            for j in range(_PQ // 16):
                rows0[r, pl.ds(j * 16, 16)] = jnp.zeros((16,), jnp.float32)
            return carry
        lax.fori_loop(0, _K, _zrow, 0)
        rows_per_tile = _NP // 16
        for b in range(rows_per_tile // _K):
            pltpu.sync_copy(rows0,
                            acc.at[pl.ds(s * rows_per_tile + b * _K, _K)])
        plsc.subcore_barrier()

        _start(0, idx0, rows0, sem0)

        def _pair(g, carry):
            _start(2 * g + 1, idx1, rows1, sem1)
            _finish(idx0, rows0, sem0)
            _start(2 * g + 2, idx0, rows0, sem0)
            _finish(idx1, rows1, sem1)
            return carry
        lax.fori_loop(0, (n_batches - 1) // 2, _pair, 0)
        _finish(idx0, rows0, sem0)
        plsc.subcore_barrier()

        pltpu.sync_copy(
            acc.at[pl.ds(s * rows_per_tile, rows_per_tile)],
            m_hbm.at[pl.ds(qbase + s * rows_per_tile, rows_per_tile)])
        plsc.subcore_barrier()


def _final_body(m_ref, h_ref, x_ref, wff2_ref, bff2_ref, o_ref):
    m4 = m_ref[...]
    cnt = m4[0, :, _HID + _MIX:_HID + _MIX + 1]
    acc = jnp.zeros(h_ref.shape, jnp.float32)
    for q in range(_MIX):
        aggq = _gelu(m4[q, :, :_HID])
        sq = m4[0, :, _HID + q:_HID + q + 1]
        acc = acc + aggq * sq
    agg2 = acc / jnp.clip(cnt, 1.0, None)
    z = _gelu(jnp.concatenate([agg2, h_ref[...]], axis=1))
    o_ref[...] = (jnp.dot(z, wff2_ref[...], preferred_element_type=jnp.float32)
                  + bff2_ref[...] + x_ref[...])


def _build(n, e, in_dim, out_dim, bn):
    assert n % bn == 0

    node = pl.pallas_call(
        _node_body,
        grid=(n // bn,),
        in_specs=[
            pl.BlockSpec((bn, in_dim), lambda i: (i, 0)),
            pl.BlockSpec((in_dim, _HID), lambda i: (0, 0)),
            pl.BlockSpec((1, _HID), lambda i: (0, 0)),
            pl.BlockSpec((_HID, _HID), lambda i: (0, 0)),
            pl.BlockSpec((_HID, _MIX), lambda i: (0, 0)),
        ],
        out_specs=[
            pl.BlockSpec((bn, _HID), lambda i: (i, 0)),
            pl.BlockSpec((_MIX, bn, _PQ), lambda i: (0, i, 0)),
        ],
        out_shape=[
            jax.ShapeDtypeStruct((n, _HID), jnp.float32),
            jax.ShapeDtypeStruct((_MIX, _NP, _PQ), jnp.float32),
        ],
    )

    mesh = plsc.VectorSubcoreMesh(
        core_axis_name="c", subcore_axis_name="s", num_cores=2)
    scatter = pl.kernel(
        _sc_scatter_body,
        mesh=mesh,
        compiler_params=pltpu.CompilerParams(use_tc_tiling_on_sc=False),
        out_type=jax.ShapeDtypeStruct((_MIX * _NP, _PQ), jnp.float32),
        scratch_types=[
            pltpu.VMEM((2, _K), jnp.int32),
            pltpu.VMEM((2, _K), jnp.int32),
            pltpu.VMEM((_K, _PQ), jnp.float32),
            pltpu.VMEM((_K, _PQ), jnp.float32),
            pltpu.VMEM_SHARED((_NP, _PQ), jnp.float32),
            pltpu.SemaphoreType.DMA,
            pltpu.SemaphoreType.DMA,
        ],
    )

    final = pl.pallas_call(
        _final_body,
        grid=(n // bn,),
        in_specs=[
            pl.BlockSpec((_MIX, bn, _PQ), lambda i: (0, i, 0)),
            pl.BlockSpec((bn, _HID), lambda i: (i, 0)),
            pl.BlockSpec((bn, in_dim), lambda i: (i, 0)),
            pl.BlockSpec((2 * _HID, out_dim), lambda i: (0, 0)),
            pl.BlockSpec((1, out_dim), lambda i: (0, 0)),
        ],
        out_specs=pl.BlockSpec((bn, out_dim), lambda i: (i, 0)),
        out_shape=jax.ShapeDtypeStruct((n, out_dim), jnp.float32),
    )

    def run(X, edge_index, W_ff1, b_ff1, W_A, W_B, W_ff2, b_ff2):
        h, cq = node(X, W_ff1, b_ff1.reshape(1, _HID), W_A, W_B)
        m = scatter(cq.reshape(_MIX * _NP, _PQ), edge_index)
        return final(m.reshape(_MIX, _NP, _PQ), h, X,
                     W_ff2, b_ff2.reshape(1, out_dim))

    return run


@jax.jit
def kernel(X, edge_index, W_ff1, b_ff1, W_A, W_B, W_ff2, b_ff2):
    run = _build(10000, 160000, 128, 128, bn=1000)
    return run(X, edge_index, W_ff1, b_ff1, W_A, W_B, W_ff2, b_ff2)
